# baseline (speedup 1.0000x reference)
import functools

import jax
import jax.numpy as jnp
from jax import lax
from jax.experimental import pallas as pl
from jax.experimental.pallas import tpu as pltpu
from jax.experimental.pallas import tpu_sc as plsc

_T = 2048
_D = 768
_E = 8
_F = 3072
_C = 512

_NC = 2
_NS = 16
_NW = _NC * _NS
_TPW = _T // _NW

_FB = 3072
_NFB = _F // _FB


def _router_body(x_ref, wrt_ref, lt_ref, eidx_ref, sidx_ref,
                 gidx_ref, gscale_ref):
    x = x_ref[...]
    wrt = wrt_ref[...]
    logits = lax.dot_general(x, wrt, (((1,), (1,)), ((), ())),
                             preferred_element_type=jnp.float32)
    lt = logits.T
    lt_ref[...] = lt

    m = jnp.max(lt, axis=0, keepdims=True)
    iota_e = lax.broadcasted_iota(jnp.int32, (_E, _T), 0)
    eidx = jnp.min(jnp.where(lt == m, iota_e, _E), axis=0,
                   keepdims=True)
    eidx_ref[...] = eidx.reshape(_T)

    p = 1.0 / jnp.sum(jnp.exp(lt - m), axis=0, keepdims=True)

    onehot = (iota_e == eidx).astype(jnp.float32)
    cum = onehot
    k = 1
    while k < _T:
        shifted = jnp.concatenate(
            [jnp.zeros((_E, k), jnp.float32), cum[:, :_T - k]], axis=1)
        cum = cum + shifted
        k *= 2
    loc = jnp.sum((cum - 1.0) * onehot, axis=0, keepdims=True)
    kept = loc < float(_C)
    loc_i = loc.astype(jnp.int32)
    slot = eidx * _C + loc_i

    sidx_ref[...] = jnp.where(kept, slot, _E * _C).reshape(_T)
    gidx_ref[...] = jnp.where(kept, slot, _E * _C).reshape(_T)
    gsc = jnp.where(kept, p, 0.0).reshape(_T, 1)
    gscale_ref[...] = jnp.broadcast_to(gsc, (_T, 128))


_router_call = pl.pallas_call(
    _router_body,
    out_shape=(
        jax.ShapeDtypeStruct((_E, _T), jnp.float32),
        jax.ShapeDtypeStruct((_T,), jnp.int32),
        jax.ShapeDtypeStruct((_T,), jnp.int32),
        jax.ShapeDtypeStruct((_T,), jnp.int32),
        jax.ShapeDtypeStruct((_T, 128), jnp.float32),
    ),
)


def _dispatch_body(flat_hbm, sidx_hbm, gs_hbm, ebuf_hbm, sbuf_hbm,
                   idx_v, rows_v, gs_v, sem, sem2, sem3):
    wid = lax.axis_index("s") * _NC + lax.axis_index("c")
    base = wid * _TPW
    half = _TPW // 2
    ld1a = pltpu.async_copy(sidx_hbm.at[pl.ds(base, half)], idx_v.at[0], sem)
    ld1b = pltpu.async_copy(sidx_hbm.at[pl.ds(base + half, half)],
                            idx_v.at[1], sem)
    ld2a = pltpu.async_copy(flat_hbm.at[pl.ds(base, half)],
                            rows_v.at[pl.ds(0, half)], sem2)
    ld2b = pltpu.async_copy(flat_hbm.at[pl.ds(base + half, half)],
                            rows_v.at[pl.ds(half, half)], sem3)
    ld3 = pltpu.async_copy(gs_hbm.at[pl.ds(base, _TPW)], gs_v, sem)
    ld1a.wait()
    ld1b.wait()
    ld2a.wait()
    cp1 = pltpu.async_copy(rows_v.at[pl.ds(0, half)],
                           ebuf_hbm.at[idx_v.at[0]], sem2)
    ld2b.wait()
    cp2 = pltpu.async_copy(rows_v.at[pl.ds(half, half)],
                           ebuf_hbm.at[idx_v.at[1]], sem3)
    ld3.wait()
    cp1.wait()
    cp2.wait()
    pltpu.async_copy(gs_v.at[pl.ds(0, half)],
                     sbuf_hbm.at[idx_v.at[0]], sem).wait()
    pltpu.async_copy(gs_v.at[pl.ds(half, half)],
                     sbuf_hbm.at[idx_v.at[1]], sem).wait()


@functools.cache
def _dispatch_call():
    return functools.partial(
        pl.kernel,
        out_type=(
            jax.ShapeDtypeStruct((_E * _C + 8, _D), jnp.float32),
            jax.ShapeDtypeStruct((_E * _C + 8, 128), jnp.float32),
        ),
        mesh=plsc.VectorSubcoreMesh(core_axis_name="c", subcore_axis_name="s"),
        scratch_types=[
            pltpu.VMEM((2, _TPW // 2), jnp.int32),
            pltpu.VMEM((_TPW, _D), jnp.float32),
            pltpu.VMEM((_TPW, 128), jnp.float32),
            pltpu.SemaphoreType.DMA,
            pltpu.SemaphoreType.DMA,
            pltpu.SemaphoreType.DMA,
        ],
    )(_dispatch_body)


def _ffn_body(x_ref, w1_ref, w2_ref, ss_ref, y_ref):
    e = pl.program_id(0)

    @pl.when(e < _E)
    def _():
        x = x_ref[...].astype(jnp.bfloat16)
        h = jnp.maximum(
            lax.dot_general(x, w1_ref[0].astype(jnp.bfloat16),
                            (((1,), (0,)), ((), ())),
                            preferred_element_type=jnp.float32), 0.0)
        y = lax.dot_general(h.astype(jnp.bfloat16),
                            w2_ref[0].astype(jnp.bfloat16),
                            (((1,), (0,)), ((), ())),
                            preferred_element_type=jnp.float32)
        y_ref[...] = y * ss_ref[:, 0:1]

    @pl.when(e == _E)
    def _():
        y_ref[...] = jnp.zeros((_C, _D), jnp.float32)


def _clampe(e):
    return jnp.minimum(e, _E - 1)


_ffn_call = pl.pallas_call(
    _ffn_body,
    grid=(_E + 1,),
    in_specs=[
        pl.BlockSpec((_C, _D), lambda e: (_clampe(e), 0)),
        pl.BlockSpec((1, _D, _F), lambda e: (_clampe(e), 0, 0)),
        pl.BlockSpec((1, _F, _D), lambda e: (_clampe(e), 0, 0)),
        pl.BlockSpec((_C, 128), lambda e: (_clampe(e), 0)),
    ],
    out_specs=pl.BlockSpec((_C, _D), lambda e: (e, 0)),
    out_shape=jax.ShapeDtypeStruct(((_E + 1) * _C, _D), jnp.float32),
    compiler_params=pltpu.CompilerParams(
        dimension_semantics=("arbitrary",)),
)


def _combine_body(y_hbm, gidx_hbm, ygath_hbm, idx_v, rows_v, sem, sem2):
    wid = lax.axis_index("s") * _NC + lax.axis_index("c")
    base = wid * _TPW
    half = _TPW // 2
    pltpu.sync_copy(gidx_hbm.at[pl.ds(base, _TPW)], idx_v)
    g0 = pltpu.async_copy(y_hbm.at[idx_v.at[pl.ds(0, half)]],
                          rows_v.at[pl.ds(0, half)], sem)
    g1 = pltpu.async_copy(y_hbm.at[idx_v.at[pl.ds(half, half)]],
                          rows_v.at[pl.ds(half, half)], sem2)
    g0.wait()
    w0 = pltpu.async_copy(rows_v.at[pl.ds(0, half)],
                          ygath_hbm.at[pl.ds(base, half)], sem)
    g1.wait()
    w1 = pltpu.async_copy(rows_v.at[pl.ds(half, half)],
                          ygath_hbm.at[pl.ds(base + half, half)], sem2)
    w0.wait()
    w1.wait()


@functools.cache
def _combine_call():
    return functools.partial(
        pl.kernel,
        out_type=jax.ShapeDtypeStruct((_T, _D), jnp.float32),
        mesh=plsc.VectorSubcoreMesh(core_axis_name="c", subcore_axis_name="s"),
        scratch_types=[
            pltpu.VMEM((_TPW,), jnp.int32),
            pltpu.VMEM((_TPW, _D), jnp.float32),
            pltpu.SemaphoreType.DMA,
            pltpu.SemaphoreType.DMA,
        ],
    )(_combine_body)


def kernel(hidden_states, w_router, w1, w2):
    B, S, D = hidden_states.shape
    flat = hidden_states.reshape(B * S, D)

    lt, eidx, sidx, gidx, gscale = _router_call(flat, w_router.T)
    ebuf, sbuf = _dispatch_call()(flat, sidx, gscale)
    y = _ffn_call(ebuf, w1, w2, sbuf)
    out = _combine_call()(y, gidx)

    return (out.reshape(B, S, D),
            (lt.T.reshape(B, S, _E), eidx.reshape(B, S)))
